# baseline (speedup 1.0000x reference)
import functools

import jax
import jax.numpy as jnp
from jax import lax
from jax.experimental import pallas as pl
from jax.experimental.pallas import tpu as pltpu
from jax.experimental.pallas import tpu_sc as plsc

_NUM_CLASSES = 20000
_G = 4
_CTX_DIM = 512
_CTX_NUM = 4
_SEQ_LEN = 77
_CTX_START = 5
_BATCH = 256


def _sc_gather(table, idx):
    info = plsc.get_sparse_core_info()
    nc, ns = info.num_cores, info.num_subcores
    nw = nc * ns
    b_per_w = _BATCH // nw
    mesh = plsc.VectorSubcoreMesh(core_axis_name="c", subcore_axis_name="s")

    @functools.partial(
        pl.kernel,
        mesh=mesh,
        out_type=jax.ShapeDtypeStruct((_BATCH, _G, _CTX_NUM, _CTX_DIM),
                                      jnp.float32),
        scratch_types=[
            pltpu.VMEM((b_per_w,), jnp.int32),
            pltpu.VMEM((b_per_w, _G, _CTX_NUM, _CTX_DIM), jnp.float32),
            pltpu.SemaphoreType.DMA,
        ],
    )
    def gather_kernel(table_hbm, idx_hbm, out_hbm, idx_v, rows_v, sem):
        wid = lax.axis_index("s") * nc + lax.axis_index("c")
        base = wid * b_per_w
        pltpu.sync_copy(idx_hbm.at[pl.ds(base, b_per_w)], idx_v)
        pltpu.async_copy(table_hbm.at[idx_v], rows_v, sem).wait()
        pltpu.sync_copy(rows_v, out_hbm.at[pl.ds(base, b_per_w)])

    return gather_kernel(table, idx)


def _tc_merge(embedding, ctx):
    bb = 4
    nbuf = 16
    nsteps = _BATCH // bb

    def body(emb_ref, ctx_ref, out_ref, bufs, sems):
        b = pl.program_id(0)

        @pl.when(b == 0)
        def _():
            bufs[...] = jnp.broadcast_to(
                emb_ref[...][None, None],
                (nbuf, bb, _G, _SEQ_LEN, _CTX_DIM))

        for k in range(nbuf):
            @pl.when(lax.rem(b, nbuf) == k)
            def _(k=k):
                @pl.when(b >= nbuf)
                def _():
                    pltpu.make_async_copy(
                        bufs.at[k],
                        out_ref.at[pl.ds((b - nbuf) * bb, bb)],
                        sems.at[k]).wait()
                bufs[k, :, :, _CTX_START:_CTX_START + _CTX_NUM, :] = (
                    ctx_ref[...])
                pltpu.make_async_copy(
                    bufs.at[k],
                    out_ref.at[pl.ds(b * bb, bb)],
                    sems.at[k]).start()

        @pl.when(b == nsteps - 1)
        def _():
            for k in range(nbuf):
                pltpu.make_async_copy(
                    bufs.at[k],
                    out_ref.at[pl.ds(b * bb, bb)],
                    sems.at[k]).wait()

    return pl.pallas_call(
        body,
        grid=(nsteps,),
        in_specs=[
            pl.BlockSpec((_G, _SEQ_LEN, _CTX_DIM), lambda b: (0, 0, 0)),
            pl.BlockSpec((bb, _G, _CTX_NUM, _CTX_DIM), lambda b: (b, 0, 0, 0)),
        ],
        out_specs=pl.BlockSpec(memory_space=pl.ANY),
        out_shape=jax.ShapeDtypeStruct((_BATCH, _G, _SEQ_LEN, _CTX_DIM),
                                       jnp.float32),
        scratch_shapes=[
            pltpu.VMEM((nbuf, bb, _G, _SEQ_LEN, _CTX_DIM), jnp.float32),
            pltpu.SemaphoreType.DMA((nbuf,)),
        ],
    )(embedding, ctx)


@jax.jit
def kernel(label, embedding, learnable_ctx):
    ctx = _sc_gather(learnable_ctx, label.astype(jnp.int32))
    return _tc_merge(embedding, ctx)


# speedup vs baseline: 1.0395x; 1.0395x over previous
import functools

import jax
import jax.numpy as jnp
from jax import lax
from jax.experimental import pallas as pl
from jax.experimental.pallas import tpu as pltpu
from jax.experimental.pallas import tpu_sc as plsc

_NUM_CLASSES = 20000
_G = 4
_CTX_DIM = 512
_CTX_NUM = 4
_SEQ_LEN = 77
_CTX_START = 5
_BATCH = 256


def _sc_gather(table, idx):
    info = plsc.get_sparse_core_info()
    nc, ns = info.num_cores, info.num_subcores
    nw = nc * ns
    b_per_w = _BATCH // nw
    mesh = plsc.VectorSubcoreMesh(core_axis_name="c", subcore_axis_name="s")

    @functools.partial(
        pl.kernel,
        mesh=mesh,
        out_type=jax.ShapeDtypeStruct((_BATCH, _G, _CTX_NUM, _CTX_DIM),
                                      jnp.float32),
        scratch_types=[
            pltpu.VMEM((b_per_w,), jnp.int32),
            pltpu.VMEM((b_per_w, _G, _CTX_NUM, _CTX_DIM), jnp.float32),
            pltpu.SemaphoreType.DMA,
        ],
    )
    def gather_kernel(table_hbm, idx_hbm, out_hbm, idx_v, rows_v, sem):
        wid = lax.axis_index("s") * nc + lax.axis_index("c")
        base = wid * b_per_w
        pltpu.sync_copy(idx_hbm.at[pl.ds(base, b_per_w)], idx_v)
        pltpu.async_copy(table_hbm.at[idx_v], rows_v, sem).wait()
        pltpu.sync_copy(rows_v, out_hbm.at[pl.ds(base, b_per_w)])

    return gather_kernel(table, idx)


def _tc_merge(embedding, ctx):
    bb = 16

    def body(emb_ref, ctx_ref, out_ref):
        out_ref[...] = jnp.broadcast_to(emb_ref[...][None],
                                        (bb, _G, _SEQ_LEN, _CTX_DIM))
        out_ref[:, :, _CTX_START:_CTX_START + _CTX_NUM, :] = ctx_ref[...]

    return pl.pallas_call(
        body,
        grid=(_BATCH // bb,),
        in_specs=[
            pl.BlockSpec((_G, _SEQ_LEN, _CTX_DIM), lambda b: (0, 0, 0)),
            pl.BlockSpec((bb, _G, _CTX_NUM, _CTX_DIM), lambda b: (b, 0, 0, 0)),
        ],
        out_specs=pl.BlockSpec((bb, _G, _SEQ_LEN, _CTX_DIM),
                               lambda b: (b, 0, 0, 0)),
        out_shape=jax.ShapeDtypeStruct((_BATCH, _G, _SEQ_LEN, _CTX_DIM),
                                       jnp.float32),
    )(embedding, ctx)


@jax.jit
def kernel(label, embedding, learnable_ctx):
    ctx = _sc_gather(learnable_ctx, label.astype(jnp.int32))
    return _tc_merge(embedding, ctx)
